# baseline (speedup 1.0000x reference)
import functools

import jax
import jax.numpy as jnp
from jax import lax
from jax.experimental import pallas as pl
from jax.experimental.pallas import tpu as pltpu
from jax.experimental.pallas import tpu_sc as plsc

N = 10000
E = 320000
D_IN = 128
H0 = 128
H1 = 64
B = 256

NC = 2
NS = 16
NW = NC * NS

EW = E // NW
CH = 80
NCHUNK = EW // CH

NPAD = 10240
DSL = NPAD // NS

ROWS_PER_W = N // NS
ZROWS = 125

POOL_CHUNK = 64
POOL_SHIFT = 6
NROWPAD = N + POOL_CHUNK
NBPAD = 10240

_mesh = plsc.VectorSubcoreMesh(core_axis_name="c", subcore_axis_name="s",
                               num_cores=NC, num_subcores=NS)


def _wid():
    return lax.axis_index("c") * NS + lax.axis_index("s")


@functools.partial(
    pl.kernel,
    out_type=jax.ShapeDtypeStruct((NC, NPAD), jnp.float32),
    mesh=_mesh,
    scratch_types=[
        pltpu.VMEM((EW,), jnp.int32),
        pltpu.VMEM((NPAD,), jnp.float32),
        pltpu.VMEM((NS, DSL), jnp.float32),
        pltpu.VMEM((DSL,), jnp.float32),
        pltpu.VMEM_SHARED((NS, NPAD), jnp.float32),
    ],
    compiler_params=pltpu.CompilerParams(needs_layout_passes=False,
                                         use_tc_tiling_on_sc=False),
)
def _deg_kernel(dst_hbm, zeros_hbm, out_hbm, dstbuf, hist, tmp, acc, parts):
    cid = lax.axis_index("c")
    sid = lax.axis_index("s")
    wid = cid * NS + sid
    pltpu.sync_copy(zeros_hbm, hist)
    pltpu.sync_copy(dst_hbm.at[pl.ds(wid * EW, EW)], dstbuf)
    ones = jnp.full((16,), 1.0, jnp.float32)

    def step(t, carry):
        idx = dstbuf[pl.ds(t * 16, 16)]
        plsc.addupdate_scatter(hist, [idx], ones)
        return carry

    lax.fori_loop(0, EW // 16, step, 0)
    pltpu.sync_copy(hist, parts.at[sid])
    plsc.subcore_barrier()
    pltpu.sync_copy(parts.at[:, pl.ds(sid * DSL, DSL)], tmp)

    def addstep(j, carry):
        sl = pl.ds(j * 16, 16)
        s = tmp[0, sl]
        for p in range(1, NS):
            s = s + tmp[p, sl]
        acc[sl] = s
        return carry

    lax.fori_loop(0, DSL // 16, addstep, 0)
    pltpu.sync_copy(acc, out_hbm.at[cid, pl.ds(sid * DSL, DSL)])


@functools.partial(
    pl.kernel,
    out_type=jax.ShapeDtypeStruct((NC, N, H1), jnp.float32),
    mesh=_mesh,
    scratch_types=[
        pltpu.VMEM((NCHUNK, CH), jnp.int32),
        pltpu.VMEM((NCHUNK, CH), jnp.int32),
        pltpu.VMEM((2, CH, H1), jnp.float32),
        pltpu.VMEM_SHARED((N, H1), jnp.float32),
        pltpu.SemaphoreType.DMA,
        pltpu.SemaphoreType.DMA,
        pltpu.SemaphoreType.DMA,
        pltpu.SemaphoreType.DMA,
    ],
    compiler_params=pltpu.CompilerParams(use_tc_tiling_on_sc=False),
)
def _agg_kernel(g_hbm, src_hbm, dst_hbm, zeros2_hbm, out_hbm,
                srcbuf, dstbuf, rows, accum, sem_a, sem_b, sem_sa, sem_sb):
    cid = lax.axis_index("c")
    sid = lax.axis_index("s")
    wid = cid * NS + sid
    for k in range(ROWS_PER_W // ZROWS):
        pltpu.sync_copy(zeros2_hbm, accum.at[pl.ds(sid * ROWS_PER_W + k * ZROWS, ZROWS)])
    plsc.subcore_barrier()
    rbase = wid * NCHUNK
    pltpu.sync_copy(src_hbm.at[pl.ds(rbase, NCHUNK)], srcbuf)
    pltpu.sync_copy(dst_hbm.at[pl.ds(rbase, NCHUNK)], dstbuf)
    rows_a = rows.at[0]
    rows_b = rows.at[1]

    def gissue(t, buf, sem):
        pltpu.async_copy(g_hbm.at[srcbuf.at[t]], buf, sem)

    def gwait(t, buf, sem):
        pltpu.make_async_copy(g_hbm.at[srcbuf.at[t]], buf, sem).wait()

    def sissue(t, buf, sem):
        pltpu.async_copy(buf, accum.at[dstbuf.at[t]], sem, add=True)

    def swait(t, buf, sem):
        pltpu.make_async_copy(buf, accum.at[dstbuf.at[t]], sem).wait()

    gissue(0, rows_a, sem_a)
    gissue(1, rows_b, sem_b)

    def body(i, carry):
        ta = 2 * i
        tb = 2 * i + 1
        gwait(ta, rows_a, sem_a)
        sissue(ta, rows_a, sem_sa)
        gwait(tb, rows_b, sem_b)
        sissue(tb, rows_b, sem_sb)
        swait(ta, rows_a, sem_sa)
        gissue(ta + 2, rows_a, sem_a)
        swait(tb, rows_b, sem_sb)
        gissue(tb + 2, rows_b, sem_b)
        return carry

    lax.fori_loop(0, (NCHUNK - 1) // 2 - 1, body, 0)
    t0 = NCHUNK - 3
    t1 = NCHUNK - 2
    t2 = NCHUNK - 1
    gwait(t0, rows_a, sem_a)
    sissue(t0, rows_a, sem_sa)
    gwait(t1, rows_b, sem_b)
    sissue(t1, rows_b, sem_sb)
    swait(t0, rows_a, sem_sa)
    gissue(t2, rows_a, sem_a)
    gwait(t2, rows_a, sem_a)
    sissue(t2, rows_a, sem_sa)
    swait(t1, rows_b, sem_sb)
    swait(t2, rows_a, sem_sa)
    plsc.subcore_barrier()
    pltpu.sync_copy(accum.at[pl.ds(sid * ROWS_PER_W, ROWS_PER_W)],
                    out_hbm.at[cid, pl.ds(sid * ROWS_PER_W, ROWS_PER_W)])


@functools.partial(
    pl.kernel,
    out_type=jax.ShapeDtypeStruct((B, 2 * H1), jnp.float32),
    mesh=_mesh,
    scratch_types=[
        pltpu.VMEM((NW, 16), jnp.int32),
        pltpu.VMEM((POOL_CHUNK, H1), jnp.float32),
        pltpu.VMEM((8, H1), jnp.float32),
        pltpu.VMEM((8, H1), jnp.float32),
        pltpu.VMEM((8, 2 * H1), jnp.float32),
    ],
    compiler_params=pltpu.CompilerParams(use_tc_tiling_on_sc=False),
)
def _pool_kernel(h_hbm, starts_hbm, zeros2_hbm, neginf_hbm, out_hbm,
                 startsbuf, rowbuf, sums, maxs, outbuf):
    wid = _wid()
    seg0 = wid * (B // NW)
    pltpu.sync_copy(starts_hbm, startsbuf)
    pltpu.sync_copy(zeros2_hbm.at[pl.ds(0, 8)], sums)
    pltpu.sync_copy(neginf_hbm, maxs)
    svec = startsbuf[wid, pl.ds(0, 16)]
    st = [svec[k] for k in range(9)]

    for k in range(8):
        lo = st[k]
        hi = st[k + 1]
        nch = lax.shift_right_logical(hi - lo + jnp.int32(POOL_CHUNK - 1),
                                      jnp.int32(POOL_SHIFT))

        def chunk_body(c, carry, lo=lo, hi=hi, k=k):
            r = lo + c * jnp.int32(POOL_CHUNK)
            pltpu.sync_copy(h_hbm.at[pl.ds(r, POOL_CHUNK)], rowbuf)
            n = jnp.minimum(jnp.int32(POOL_CHUNK), hi - r)

            def row_step(j, inner):
                for q in range(H1 // 16):
                    sl = pl.ds(q * 16, 16)
                    v = rowbuf[j, sl]
                    sums[k, sl] = sums[k, sl] + v
                    maxs[k, sl] = jnp.maximum(maxs[k, sl], v)
                return inner

            lax.fori_loop(0, n, row_step, 0)
            return carry

        lax.fori_loop(0, nch, chunk_body, 0)

    for k in range(8):
        cnt = st[k + 1] - st[k]
        cntv = jnp.full((16,), cnt.astype(jnp.float32))
        invv = jnp.full((16,), 1.0, jnp.float32) / jnp.maximum(cntv, 1.0)
        nonempty = cnt > 0
        for q in range(H1 // 16):
            sl = pl.ds(q * 16, 16)
            outbuf[k, sl] = sums[k, sl] * invv
            outbuf[k, pl.ds(H1 + q * 16, 16)] = jnp.where(
                nonempty, maxs[k, sl], jnp.float32(0.0))
    pltpu.sync_copy(outbuf, out_hbm.at[pl.ds(seg0, 8)])


def _embed_scale_body(x_ref, we_ref, be_ref, wg_ref, d0_ref, d1_ref,
                      g_ref, dinvb_ref):
    h0 = jnp.dot(x_ref[...], we_ref[...], preferred_element_type=jnp.float32)
    h0 = jnp.maximum(h0 + be_ref[...], 0.0)
    h1 = jnp.dot(h0, wg_ref[...], preferred_element_type=jnp.float32)
    deg = d0_ref[...] + d1_ref[...] + 1.0
    dinv = lax.rsqrt(deg)
    dinvb = jnp.broadcast_to(dinv, h1.shape)
    g_ref[...] = h1 * dinvb
    dinvb_ref[...] = dinvb


def _bn_body(a0_ref, a1_ref, g_ref, dinvb_ref, bg_ref, gamma_ref, beta_ref,
             b2d_ref, out_ref, starts_ref):
    y = dinvb_ref[...] * (a0_ref[...] + a1_ref[...] + g_ref[...]) + bg_ref[...]
    mu = jnp.mean(y, axis=0, keepdims=True)
    var = jnp.mean((y - mu) ** 2, axis=0, keepdims=True)
    hbn = gamma_ref[...] * (y - mu) * lax.rsqrt(var + 1e-5) + beta_ref[...]
    out_ref[0:N, :] = jnp.maximum(hbn, 0.0)
    out_ref[N:NROWPAD, :] = jnp.zeros((NROWPAD - N, H1), jnp.float32)
    sids = lax.broadcasted_iota(jnp.int32, (B, 1), 0)
    cnt = jnp.zeros((B,), jnp.float32)
    rows_per_chunk = 20
    for c in range(NBPAD // (rows_per_chunk * 128)):
        blk = b2d_ref[c * rows_per_chunk:(c + 1) * rows_per_chunk, :]
        bflat = blk.reshape(1, rows_per_chunk * 128)
        cnt = cnt + jnp.sum((bflat < sids).astype(jnp.float32), axis=1)
    cnt_lt = cnt.astype(jnp.int32)
    s_main = cnt_lt.reshape(NW, 8)
    s_end = jnp.concatenate(
        [s_main[1:NW, 0:1], jnp.full((1, 1), N, jnp.int32)], axis=0)
    pad = jnp.zeros((NW, 7), jnp.int32)
    starts_ref[...] = jnp.concatenate([s_main, s_end, pad], axis=1)


def _fc_body(p_ref, w1_ref, b1_ref, w2_ref, b2_ref, out_ref):
    t = jnp.dot(p_ref[...], w1_ref[...], preferred_element_type=jnp.float32)
    t = jnp.maximum(t + b1_ref[...], 0.0)
    out_ref[...] = (jnp.dot(t, w2_ref[...], preferred_element_type=jnp.float32)
                    + b2_ref[...])


ROW_BLK = 400
NBLK = N // ROW_BLK


def kernel(x, edge_index, edge_attr, batch, W_emb, b_emb, W_g, b_g, gamma,
           beta, W_fc1, b_fc1, W_fc2, b_fc2):
    del edge_attr
    src = edge_index[0]
    dst = edge_index[1]
    zeros1 = jnp.zeros((NPAD,), jnp.float32)
    zeros2 = jnp.zeros((ZROWS, H1), jnp.float32)
    neginf = jnp.full((8, H1), -jnp.inf, jnp.float32)

    deg_parts = _deg_kernel(dst, zeros1)
    d0 = deg_parts[0, :N].reshape(N, 1)
    d1 = deg_parts[1, :N].reshape(N, 1)

    g, dinvb = pl.pallas_call(
        _embed_scale_body,
        grid=(NBLK,),
        in_specs=[
            pl.BlockSpec((ROW_BLK, D_IN), lambda i: (i, 0)),
            pl.BlockSpec((D_IN, H0), lambda i: (0, 0)),
            pl.BlockSpec((1, H0), lambda i: (0, 0)),
            pl.BlockSpec((H0, H1), lambda i: (0, 0)),
            pl.BlockSpec((ROW_BLK, 1), lambda i: (i, 0)),
            pl.BlockSpec((ROW_BLK, 1), lambda i: (i, 0)),
        ],
        out_specs=[
            pl.BlockSpec((ROW_BLK, H1), lambda i: (i, 0)),
            pl.BlockSpec((ROW_BLK, H1), lambda i: (i, 0)),
        ],
        out_shape=[
            jax.ShapeDtypeStruct((N, H1), jnp.float32),
            jax.ShapeDtypeStruct((N, H1), jnp.float32),
        ],
    )(x, W_emb, b_emb.reshape(1, H0), W_g, d0, d1)

    agg = _agg_kernel(g, src.reshape(NW * NCHUNK, CH),
                      dst.reshape(NW * NCHUNK, CH), zeros2)

    b2d = jnp.concatenate(
        [batch, jnp.full((NBPAD - N,), B + 44, jnp.int32)]).reshape(NBPAD // 128, 128)
    hbn, starts = pl.pallas_call(
        _bn_body,
        out_shape=[
            jax.ShapeDtypeStruct((NROWPAD, H1), jnp.float32),
            jax.ShapeDtypeStruct((NW, 16), jnp.int32),
        ],
    )(agg[0], agg[1], g, dinvb, b_g.reshape(1, H1), gamma.reshape(1, H1),
      beta.reshape(1, H1), b2d)

    pooled = _pool_kernel(hbn, starts, zeros2, neginf)

    out = pl.pallas_call(
        _fc_body,
        out_shape=jax.ShapeDtypeStruct((B, 128), jnp.float32),
    )(pooled, W_fc1, b_fc1.reshape(1, 1024), W_fc2, b_fc2.reshape(1, 128))
    return out


# speedup vs baseline: 1.0640x; 1.0640x over previous
import functools

import jax
import jax.numpy as jnp
from jax import lax
from jax.experimental import pallas as pl
from jax.experimental.pallas import tpu as pltpu
from jax.experimental.pallas import tpu_sc as plsc

N = 10000
E = 320000
D_IN = 128
H0 = 128
H1 = 64
B = 256

NC = 2
NS = 16
NW = NC * NS

EW = E // NW
CH = 80
NCHUNK = EW // CH

NPAD = 10240
DSL = NPAD // NS

ROWS_PER_W = N // NS
ZROWS = 125

POOL_CHUNK = 64
POOL_SHIFT = 6
NROWPAD = N + POOL_CHUNK
NBPAD = 10240

_mesh = plsc.VectorSubcoreMesh(core_axis_name="c", subcore_axis_name="s",
                               num_cores=NC, num_subcores=NS)


def _wid():
    return lax.axis_index("c") * NS + lax.axis_index("s")


@functools.partial(
    pl.kernel,
    out_type=jax.ShapeDtypeStruct((NC, NPAD), jnp.float32),
    mesh=_mesh,
    scratch_types=[
        pltpu.VMEM((EW,), jnp.int32),
        pltpu.VMEM((NPAD,), jnp.float32),
        pltpu.VMEM((NS, DSL), jnp.float32),
        pltpu.VMEM((DSL,), jnp.float32),
        pltpu.VMEM_SHARED((NS, NPAD), jnp.float32),
    ],
    compiler_params=pltpu.CompilerParams(needs_layout_passes=False,
                                         use_tc_tiling_on_sc=False),
)
def _deg_kernel(dst_hbm, zeros_hbm, out_hbm, dstbuf, hist, tmp, acc, parts):
    cid = lax.axis_index("c")
    sid = lax.axis_index("s")
    wid = cid * NS + sid
    pltpu.sync_copy(zeros_hbm, hist)
    pltpu.sync_copy(dst_hbm.at[pl.ds(wid * EW, EW)], dstbuf)
    ones = jnp.full((16,), 1.0, jnp.float32)

    def step(t, carry):
        idx = dstbuf[pl.ds(t * 16, 16)]
        plsc.addupdate_scatter(hist, [idx], ones)
        return carry

    lax.fori_loop(0, EW // 16, step, 0)
    pltpu.sync_copy(hist, parts.at[sid])
    plsc.subcore_barrier()
    pltpu.sync_copy(parts.at[:, pl.ds(sid * DSL, DSL)], tmp)

    def addstep(j, carry):
        sl = pl.ds(j * 16, 16)
        s = tmp[0, sl]
        for p in range(1, NS):
            s = s + tmp[p, sl]
        acc[sl] = s
        return carry

    lax.fori_loop(0, DSL // 16, addstep, 0)
    pltpu.sync_copy(acc, out_hbm.at[cid, pl.ds(sid * DSL, DSL)])


@functools.partial(
    pl.kernel,
    out_type=jax.ShapeDtypeStruct((NC, N, H1), jnp.float32),
    mesh=_mesh,
    scratch_types=[
        pltpu.VMEM((NCHUNK, CH), jnp.int32),
        pltpu.VMEM((NCHUNK, CH), jnp.int32),
        pltpu.VMEM((2, CH, H1), jnp.float32),
        pltpu.VMEM_SHARED((N, H1), jnp.float32),
        pltpu.SemaphoreType.DMA,
        pltpu.SemaphoreType.DMA,
    ],
    compiler_params=pltpu.CompilerParams(use_tc_tiling_on_sc=False),
)
def _agg_kernel(g_hbm, src_hbm, dst_hbm, zeros2_hbm, out_hbm,
                srcbuf, dstbuf, rows, accum, sem_a, sem_b):
    cid = lax.axis_index("c")
    sid = lax.axis_index("s")
    wid = cid * NS + sid
    for k in range(ROWS_PER_W // ZROWS):
        pltpu.sync_copy(zeros2_hbm, accum.at[pl.ds(sid * ROWS_PER_W + k * ZROWS, ZROWS)])
    plsc.subcore_barrier()
    rbase = wid * NCHUNK
    pltpu.sync_copy(src_hbm.at[pl.ds(rbase, NCHUNK)], srcbuf)
    pltpu.sync_copy(dst_hbm.at[pl.ds(rbase, NCHUNK)], dstbuf)
    rows_a = rows.at[0]
    rows_b = rows.at[1]

    def gissue(t, buf, sem):
        pltpu.async_copy(g_hbm.at[srcbuf.at[t]], buf, sem)

    def gwait(t, buf, sem):
        pltpu.make_async_copy(g_hbm.at[srcbuf.at[t]], buf, sem).wait()

    def scatter(t, buf):
        pltpu.sync_copy(buf, accum.at[dstbuf.at[t]], add=True)

    gissue(0, rows_a, sem_a)

    def body(i, carry):
        ta = 2 * i
        tb = 2 * i + 1
        gissue(tb, rows_b, sem_b)
        gwait(ta, rows_a, sem_a)
        scatter(ta, rows_a)
        gissue(ta + 2, rows_a, sem_a)
        gwait(tb, rows_b, sem_b)
        scatter(tb, rows_b)
        return carry

    lax.fori_loop(0, (NCHUNK - 1) // 2, body, 0)
    tlast = NCHUNK - 1
    gwait(tlast, rows_a, sem_a)
    scatter(tlast, rows_a)
    plsc.subcore_barrier()
    pltpu.sync_copy(accum.at[pl.ds(sid * ROWS_PER_W, ROWS_PER_W)],
                    out_hbm.at[cid, pl.ds(sid * ROWS_PER_W, ROWS_PER_W)])


@functools.partial(
    pl.kernel,
    out_type=jax.ShapeDtypeStruct((B, 2 * H1), jnp.float32),
    mesh=_mesh,
    scratch_types=[
        pltpu.VMEM((NW, 16), jnp.int32),
        pltpu.VMEM((POOL_CHUNK, H1), jnp.float32),
        pltpu.VMEM((8, H1), jnp.float32),
        pltpu.VMEM((8, H1), jnp.float32),
        pltpu.VMEM((8, 2 * H1), jnp.float32),
    ],
    compiler_params=pltpu.CompilerParams(use_tc_tiling_on_sc=False),
)
def _pool_kernel(h_hbm, starts_hbm, zeros2_hbm, neginf_hbm, out_hbm,
                 startsbuf, rowbuf, sums, maxs, outbuf):
    wid = _wid()
    seg0 = wid * (B // NW)
    pltpu.sync_copy(starts_hbm, startsbuf)
    pltpu.sync_copy(zeros2_hbm.at[pl.ds(0, 8)], sums)
    pltpu.sync_copy(neginf_hbm, maxs)
    svec = startsbuf[wid, pl.ds(0, 16)]
    st = [svec[k] for k in range(9)]

    for k in range(8):
        lo = st[k]
        hi = st[k + 1]
        nch = lax.shift_right_logical(hi - lo + jnp.int32(POOL_CHUNK - 1),
                                      jnp.int32(POOL_SHIFT))

        def chunk_body(c, carry, lo=lo, hi=hi, k=k):
            r = lo + c * jnp.int32(POOL_CHUNK)
            pltpu.sync_copy(h_hbm.at[pl.ds(r, POOL_CHUNK)], rowbuf)
            n = jnp.minimum(jnp.int32(POOL_CHUNK), hi - r)

            def row_step(j, inner):
                for q in range(H1 // 16):
                    sl = pl.ds(q * 16, 16)
                    v = rowbuf[j, sl]
                    sums[k, sl] = sums[k, sl] + v
                    maxs[k, sl] = jnp.maximum(maxs[k, sl], v)
                return inner

            lax.fori_loop(0, n, row_step, 0)
            return carry

        lax.fori_loop(0, nch, chunk_body, 0)

    for k in range(8):
        cnt = st[k + 1] - st[k]
        cntv = jnp.full((16,), cnt.astype(jnp.float32))
        invv = jnp.full((16,), 1.0, jnp.float32) / jnp.maximum(cntv, 1.0)
        nonempty = cnt > 0
        for q in range(H1 // 16):
            sl = pl.ds(q * 16, 16)
            outbuf[k, sl] = sums[k, sl] * invv
            outbuf[k, pl.ds(H1 + q * 16, 16)] = jnp.where(
                nonempty, maxs[k, sl], jnp.float32(0.0))
    pltpu.sync_copy(outbuf, out_hbm.at[pl.ds(seg0, 8)])


def _embed_scale_body(x_ref, we_ref, be_ref, wg_ref, d0_ref, d1_ref,
                      g_ref, dinvb_ref):
    h0 = jnp.dot(x_ref[...], we_ref[...], preferred_element_type=jnp.float32)
    h0 = jnp.maximum(h0 + be_ref[...], 0.0)
    h1 = jnp.dot(h0, wg_ref[...], preferred_element_type=jnp.float32)
    deg = d0_ref[...] + d1_ref[...] + 1.0
    dinv = lax.rsqrt(deg)
    dinvb = jnp.broadcast_to(dinv, h1.shape)
    g_ref[...] = h1 * dinvb
    dinvb_ref[...] = dinvb


def _bn_body(a0_ref, a1_ref, g_ref, dinvb_ref, bg_ref, gamma_ref, beta_ref,
             b2d_ref, out_ref, starts_ref):
    y = dinvb_ref[...] * (a0_ref[...] + a1_ref[...] + g_ref[...]) + bg_ref[...]
    mu = jnp.mean(y, axis=0, keepdims=True)
    var = jnp.mean((y - mu) ** 2, axis=0, keepdims=True)
    hbn = gamma_ref[...] * (y - mu) * lax.rsqrt(var + 1e-5) + beta_ref[...]
    out_ref[0:N, :] = jnp.maximum(hbn, 0.0)
    out_ref[N:NROWPAD, :] = jnp.zeros((NROWPAD - N, H1), jnp.float32)
    sids = lax.broadcasted_iota(jnp.int32, (B, 1), 0)
    cnt = jnp.zeros((B,), jnp.float32)
    rows_per_chunk = 20
    for c in range(NBPAD // (rows_per_chunk * 128)):
        blk = b2d_ref[c * rows_per_chunk:(c + 1) * rows_per_chunk, :]
        bflat = blk.reshape(1, rows_per_chunk * 128)
        cnt = cnt + jnp.sum((bflat < sids).astype(jnp.float32), axis=1)
    cnt_lt = cnt.astype(jnp.int32)
    s_main = cnt_lt.reshape(NW, 8)
    s_end = jnp.concatenate(
        [s_main[1:NW, 0:1], jnp.full((1, 1), N, jnp.int32)], axis=0)
    pad = jnp.zeros((NW, 7), jnp.int32)
    starts_ref[...] = jnp.concatenate([s_main, s_end, pad], axis=1)


def _fc_body(p_ref, w1_ref, b1_ref, w2_ref, b2_ref, out_ref):
    t = jnp.dot(p_ref[...], w1_ref[...], preferred_element_type=jnp.float32)
    t = jnp.maximum(t + b1_ref[...], 0.0)
    out_ref[...] = (jnp.dot(t, w2_ref[...], preferred_element_type=jnp.float32)
                    + b2_ref[...])


ROW_BLK = 400
NBLK = N // ROW_BLK


def kernel(x, edge_index, edge_attr, batch, W_emb, b_emb, W_g, b_g, gamma,
           beta, W_fc1, b_fc1, W_fc2, b_fc2):
    del edge_attr
    src = edge_index[0]
    dst = edge_index[1]
    zeros1 = jnp.zeros((NPAD,), jnp.float32)
    zeros2 = jnp.zeros((ZROWS, H1), jnp.float32)
    neginf = jnp.full((8, H1), -jnp.inf, jnp.float32)

    deg_parts = _deg_kernel(dst, zeros1)
    d0 = deg_parts[0, :N].reshape(N, 1)
    d1 = deg_parts[1, :N].reshape(N, 1)

    g, dinvb = pl.pallas_call(
        _embed_scale_body,
        grid=(NBLK,),
        in_specs=[
            pl.BlockSpec((ROW_BLK, D_IN), lambda i: (i, 0)),
            pl.BlockSpec((D_IN, H0), lambda i: (0, 0)),
            pl.BlockSpec((1, H0), lambda i: (0, 0)),
            pl.BlockSpec((H0, H1), lambda i: (0, 0)),
            pl.BlockSpec((ROW_BLK, 1), lambda i: (i, 0)),
            pl.BlockSpec((ROW_BLK, 1), lambda i: (i, 0)),
        ],
        out_specs=[
            pl.BlockSpec((ROW_BLK, H1), lambda i: (i, 0)),
            pl.BlockSpec((ROW_BLK, H1), lambda i: (i, 0)),
        ],
        out_shape=[
            jax.ShapeDtypeStruct((N, H1), jnp.float32),
            jax.ShapeDtypeStruct((N, H1), jnp.float32),
        ],
    )(x, W_emb, b_emb.reshape(1, H0), W_g, d0, d1)

    agg = _agg_kernel(g, src.reshape(NW * NCHUNK, CH),
                      dst.reshape(NW * NCHUNK, CH), zeros2)

    b2d = jnp.concatenate(
        [batch, jnp.full((NBPAD - N,), B + 44, jnp.int32)]).reshape(NBPAD // 128, 128)
    hbn, starts = pl.pallas_call(
        _bn_body,
        out_shape=[
            jax.ShapeDtypeStruct((NROWPAD, H1), jnp.float32),
            jax.ShapeDtypeStruct((NW, 16), jnp.int32),
        ],
    )(agg[0], agg[1], g, dinvb, b_g.reshape(1, H1), gamma.reshape(1, H1),
      beta.reshape(1, H1), b2d)

    pooled = _pool_kernel(hbn, starts, zeros2, neginf)

    out = pl.pallas_call(
        _fc_body,
        out_shape=jax.ShapeDtypeStruct((B, 128), jnp.float32),
    )(pooled, W_fc1, b_fc1.reshape(1, 1024), W_fc2, b_fc2.reshape(1, 128))
    return out


# speedup vs baseline: 1.0977x; 1.0317x over previous
import functools

import jax
import jax.numpy as jnp
from jax import lax
from jax.experimental import pallas as pl
from jax.experimental.pallas import tpu as pltpu
from jax.experimental.pallas import tpu_sc as plsc

N = 10000
E = 320000
D_IN = 128
H0 = 128
H1 = 64
B = 256

NC = 2
NS = 16
NW = NC * NS

EW = E // NW
CH = 100
NCHUNK = EW // CH

NPAD = 10240
DSL = NPAD // NS

ROWS_PER_W = N // NS
ZROWS = 125

POOL_CHUNK = 64
POOL_SHIFT = 6
NROWPAD = N + POOL_CHUNK
NBPAD = 10240

_mesh = plsc.VectorSubcoreMesh(core_axis_name="c", subcore_axis_name="s",
                               num_cores=NC, num_subcores=NS)


def _wid():
    return lax.axis_index("c") * NS + lax.axis_index("s")


@functools.partial(
    pl.kernel,
    out_type=jax.ShapeDtypeStruct((NC, NPAD), jnp.float32),
    mesh=_mesh,
    scratch_types=[
        pltpu.VMEM((EW,), jnp.int32),
        pltpu.VMEM((NPAD,), jnp.float32),
        pltpu.VMEM((NS, DSL), jnp.float32),
        pltpu.VMEM((DSL,), jnp.float32),
        pltpu.VMEM_SHARED((NS, NPAD), jnp.float32),
    ],
    compiler_params=pltpu.CompilerParams(needs_layout_passes=False,
                                         use_tc_tiling_on_sc=False),
)
def _deg_kernel(dst_hbm, zeros_hbm, out_hbm, dstbuf, hist, tmp, acc, parts):
    cid = lax.axis_index("c")
    sid = lax.axis_index("s")
    wid = cid * NS + sid
    pltpu.sync_copy(zeros_hbm, hist)
    pltpu.sync_copy(dst_hbm.at[pl.ds(wid * EW, EW)], dstbuf)
    ones = jnp.full((16,), 1.0, jnp.float32)

    def step(t, carry):
        idx = dstbuf[pl.ds(t * 16, 16)]
        plsc.addupdate_scatter(hist, [idx], ones)
        return carry

    lax.fori_loop(0, EW // 16, step, 0)
    pltpu.sync_copy(hist, parts.at[sid])
    plsc.subcore_barrier()
    pltpu.sync_copy(parts.at[:, pl.ds(sid * DSL, DSL)], tmp)

    def addstep(j, carry):
        sl = pl.ds(j * 16, 16)
        s = tmp[0, sl]
        for p in range(1, NS):
            s = s + tmp[p, sl]
        acc[sl] = s
        return carry

    lax.fori_loop(0, DSL // 16, addstep, 0)
    pltpu.sync_copy(acc, out_hbm.at[cid, pl.ds(sid * DSL, DSL)])


@functools.partial(
    pl.kernel,
    out_type=jax.ShapeDtypeStruct((NC, N, H1), jnp.float32),
    mesh=_mesh,
    scratch_types=[
        pltpu.VMEM((NCHUNK, CH), jnp.int32),
        pltpu.VMEM((NCHUNK, CH), jnp.int32),
        pltpu.VMEM((2, CH, H1), jnp.float32),
        pltpu.VMEM_SHARED((N, H1), jnp.float32),
        pltpu.SemaphoreType.DMA,
        pltpu.SemaphoreType.DMA,
    ],
    compiler_params=pltpu.CompilerParams(use_tc_tiling_on_sc=False),
)
def _agg_kernel(g_hbm, src_hbm, dst_hbm, zeros2_hbm, out_hbm,
                srcbuf, dstbuf, rows, accum, sem_a, sem_b):
    cid = lax.axis_index("c")
    sid = lax.axis_index("s")
    wid = cid * NS + sid
    for k in range(ROWS_PER_W // ZROWS):
        pltpu.sync_copy(zeros2_hbm, accum.at[pl.ds(sid * ROWS_PER_W + k * ZROWS, ZROWS)])
    plsc.subcore_barrier()
    rbase = wid * NCHUNK
    pltpu.sync_copy(src_hbm.at[pl.ds(rbase, NCHUNK)], srcbuf)
    pltpu.sync_copy(dst_hbm.at[pl.ds(rbase, NCHUNK)], dstbuf)
    rows_a = rows.at[0]
    rows_b = rows.at[1]

    def gissue(t, buf, sem):
        pltpu.async_copy(g_hbm.at[srcbuf.at[t]], buf, sem)

    def gwait(t, buf, sem):
        pltpu.make_async_copy(g_hbm.at[srcbuf.at[t]], buf, sem).wait()

    def scatter(t, buf):
        pltpu.sync_copy(buf, accum.at[dstbuf.at[t]], add=True)

    gissue(0, rows_a, sem_a)
    gissue(1, rows_b, sem_b)

    def body(i, carry):
        ta = 2 * i
        tb = 2 * i + 1
        gwait(ta, rows_a, sem_a)
        scatter(ta, rows_a)
        gissue(ta + 2, rows_a, sem_a)
        gwait(tb, rows_b, sem_b)
        scatter(tb, rows_b)
        gissue(tb + 2, rows_b, sem_b)
        return carry

    lax.fori_loop(0, NCHUNK // 2 - 1, body, 0)
    ta = NCHUNK - 2
    tb = NCHUNK - 1
    gwait(ta, rows_a, sem_a)
    scatter(ta, rows_a)
    gwait(tb, rows_b, sem_b)
    scatter(tb, rows_b)
    plsc.subcore_barrier()
    pltpu.sync_copy(accum.at[pl.ds(sid * ROWS_PER_W, ROWS_PER_W)],
                    out_hbm.at[cid, pl.ds(sid * ROWS_PER_W, ROWS_PER_W)])


@functools.partial(
    pl.kernel,
    out_type=jax.ShapeDtypeStruct((B, 2 * H1), jnp.float32),
    mesh=_mesh,
    scratch_types=[
        pltpu.VMEM((NW, 16), jnp.int32),
        pltpu.VMEM((POOL_CHUNK, H1), jnp.float32),
        pltpu.VMEM((8, H1), jnp.float32),
        pltpu.VMEM((8, H1), jnp.float32),
        pltpu.VMEM((8, 2 * H1), jnp.float32),
    ],
    compiler_params=pltpu.CompilerParams(use_tc_tiling_on_sc=False),
)
def _pool_kernel(h_hbm, starts_hbm, zeros2_hbm, neginf_hbm, out_hbm,
                 startsbuf, rowbuf, sums, maxs, outbuf):
    wid = _wid()
    seg0 = wid * (B // NW)
    pltpu.sync_copy(starts_hbm, startsbuf)
    pltpu.sync_copy(zeros2_hbm.at[pl.ds(0, 8)], sums)
    pltpu.sync_copy(neginf_hbm, maxs)
    svec = startsbuf[wid, pl.ds(0, 16)]
    st = [svec[k] for k in range(9)]

    for k in range(8):
        lo = st[k]
        hi = st[k + 1]
        nch = lax.shift_right_logical(hi - lo + jnp.int32(POOL_CHUNK - 1),
                                      jnp.int32(POOL_SHIFT))

        def chunk_body(c, carry, lo=lo, hi=hi, k=k):
            r = lo + c * jnp.int32(POOL_CHUNK)
            pltpu.sync_copy(h_hbm.at[pl.ds(r, POOL_CHUNK)], rowbuf)
            n = jnp.minimum(jnp.int32(POOL_CHUNK), hi - r)

            def row_step(j, inner):
                for q in range(H1 // 16):
                    sl = pl.ds(q * 16, 16)
                    v = rowbuf[j, sl]
                    sums[k, sl] = sums[k, sl] + v
                    maxs[k, sl] = jnp.maximum(maxs[k, sl], v)
                return inner

            lax.fori_loop(0, n, row_step, 0)
            return carry

        lax.fori_loop(0, nch, chunk_body, 0)

    for k in range(8):
        cnt = st[k + 1] - st[k]
        cntv = jnp.full((16,), cnt.astype(jnp.float32))
        invv = jnp.full((16,), 1.0, jnp.float32) / jnp.maximum(cntv, 1.0)
        nonempty = cnt > 0
        for q in range(H1 // 16):
            sl = pl.ds(q * 16, 16)
            outbuf[k, sl] = sums[k, sl] * invv
            outbuf[k, pl.ds(H1 + q * 16, 16)] = jnp.where(
                nonempty, maxs[k, sl], jnp.float32(0.0))
    pltpu.sync_copy(outbuf, out_hbm.at[pl.ds(seg0, 8)])


def _embed_scale_body(x_ref, we_ref, be_ref, wg_ref, d0_ref, d1_ref,
                      g_ref, dinvb_ref):
    h0 = jnp.dot(x_ref[...], we_ref[...], preferred_element_type=jnp.float32)
    h0 = jnp.maximum(h0 + be_ref[...], 0.0)
    h1 = jnp.dot(h0, wg_ref[...], preferred_element_type=jnp.float32)
    deg = d0_ref[...] + d1_ref[...] + 1.0
    dinv = lax.rsqrt(deg)
    dinvb = jnp.broadcast_to(dinv, h1.shape)
    g_ref[...] = h1 * dinvb
    dinvb_ref[...] = dinvb


def _bn_body(a0_ref, a1_ref, g_ref, dinvb_ref, bg_ref, gamma_ref, beta_ref,
             b2d_ref, out_ref, starts_ref):
    y = dinvb_ref[...] * (a0_ref[...] + a1_ref[...] + g_ref[...]) + bg_ref[...]
    mu = jnp.mean(y, axis=0, keepdims=True)
    var = jnp.mean((y - mu) ** 2, axis=0, keepdims=True)
    hbn = gamma_ref[...] * (y - mu) * lax.rsqrt(var + 1e-5) + beta_ref[...]
    out_ref[0:N, :] = jnp.maximum(hbn, 0.0)
    out_ref[N:NROWPAD, :] = jnp.zeros((NROWPAD - N, H1), jnp.float32)
    sids = lax.broadcasted_iota(jnp.int32, (B, 1), 0)
    cnt = jnp.zeros((B,), jnp.float32)
    rows_per_chunk = 20
    for c in range(NBPAD // (rows_per_chunk * 128)):
        blk = b2d_ref[c * rows_per_chunk:(c + 1) * rows_per_chunk, :]
        bflat = blk.reshape(1, rows_per_chunk * 128)
        cnt = cnt + jnp.sum((bflat < sids).astype(jnp.float32), axis=1)
    cnt_lt = cnt.astype(jnp.int32)
    s_main = cnt_lt.reshape(NW, 8)
    s_end = jnp.concatenate(
        [s_main[1:NW, 0:1], jnp.full((1, 1), N, jnp.int32)], axis=0)
    pad = jnp.zeros((NW, 7), jnp.int32)
    starts_ref[...] = jnp.concatenate([s_main, s_end, pad], axis=1)


def _fc_body(p_ref, w1_ref, b1_ref, w2_ref, b2_ref, out_ref):
    t = jnp.dot(p_ref[...], w1_ref[...], preferred_element_type=jnp.float32)
    t = jnp.maximum(t + b1_ref[...], 0.0)
    out_ref[...] = (jnp.dot(t, w2_ref[...], preferred_element_type=jnp.float32)
                    + b2_ref[...])


ROW_BLK = 400
NBLK = N // ROW_BLK


def kernel(x, edge_index, edge_attr, batch, W_emb, b_emb, W_g, b_g, gamma,
           beta, W_fc1, b_fc1, W_fc2, b_fc2):
    del edge_attr
    src = edge_index[0]
    dst = edge_index[1]
    zeros1 = jnp.zeros((NPAD,), jnp.float32)
    zeros2 = jnp.zeros((ZROWS, H1), jnp.float32)
    neginf = jnp.full((8, H1), -jnp.inf, jnp.float32)

    deg_parts = _deg_kernel(dst, zeros1)
    d0 = deg_parts[0, :N].reshape(N, 1)
    d1 = deg_parts[1, :N].reshape(N, 1)

    g, dinvb = pl.pallas_call(
        _embed_scale_body,
        grid=(NBLK,),
        in_specs=[
            pl.BlockSpec((ROW_BLK, D_IN), lambda i: (i, 0)),
            pl.BlockSpec((D_IN, H0), lambda i: (0, 0)),
            pl.BlockSpec((1, H0), lambda i: (0, 0)),
            pl.BlockSpec((H0, H1), lambda i: (0, 0)),
            pl.BlockSpec((ROW_BLK, 1), lambda i: (i, 0)),
            pl.BlockSpec((ROW_BLK, 1), lambda i: (i, 0)),
        ],
        out_specs=[
            pl.BlockSpec((ROW_BLK, H1), lambda i: (i, 0)),
            pl.BlockSpec((ROW_BLK, H1), lambda i: (i, 0)),
        ],
        out_shape=[
            jax.ShapeDtypeStruct((N, H1), jnp.float32),
            jax.ShapeDtypeStruct((N, H1), jnp.float32),
        ],
    )(x, W_emb, b_emb.reshape(1, H0), W_g, d0, d1)

    agg = _agg_kernel(g, src.reshape(NW * NCHUNK, CH),
                      dst.reshape(NW * NCHUNK, CH), zeros2)

    b2d = jnp.concatenate(
        [batch, jnp.full((NBPAD - N,), B + 44, jnp.int32)]).reshape(NBPAD // 128, 128)
    hbn, starts = pl.pallas_call(
        _bn_body,
        out_shape=[
            jax.ShapeDtypeStruct((NROWPAD, H1), jnp.float32),
            jax.ShapeDtypeStruct((NW, 16), jnp.int32),
        ],
    )(agg[0], agg[1], g, dinvb, b_g.reshape(1, H1), gamma.reshape(1, H1),
      beta.reshape(1, H1), b2d)

    pooled = _pool_kernel(hbn, starts, zeros2, neginf)

    out = pl.pallas_call(
        _fc_body,
        out_shape=jax.ShapeDtypeStruct((B, 128), jnp.float32),
    )(pooled, W_fc1, b_fc1.reshape(1, 1024), W_fc2, b_fc2.reshape(1, 128))
    return out
